# baseline (speedup 1.0000x reference)
import functools

import jax
import jax.numpy as jnp
from jax import lax
from jax.experimental import pallas as pl
from jax.experimental.pallas import tpu as pltpu

try:
    from jax.experimental.pallas import tpu_sc as plsc
    _HAS_SC = True
except ImportError:
    plsc = None
    _HAS_SC = False

DIM = 1024
MLP = 4096
E = 8
TOPK = 2
BLK = 256
G = 16 + E
P = G * BLK

NW = 32
_NC = 2


def _lin_block(x_ref, x3_ref):
    for s in range(_SL):
        x3_ref[:, s, :] = x_ref[:, s * 128:(s + 1) * 128]


def _linearize(xf, n):
    blk = 256
    return pl.pallas_call(
        _lin_block,
        grid=(n // blk,),
        in_specs=[pl.BlockSpec((blk, DIM), lambda i: (i, 0))],
        out_specs=pl.BlockSpec((blk, _SL, 128), lambda i: (i, 0, 0)),
        out_shape=jax.ShapeDtypeStruct((n, _SL, 128), jnp.float32),
    )(xf)


def _ffn_block(be_ref, bv_ref, xg_ref, gate_ref, lng_ref, lnb_ref,
               w1_ref, b1_ref, w2_ref, b2_ref, y_ref):
    g = pl.program_id(0)

    @pl.when(bv_ref[g] != 0)
    def _():
        xg = jnp.concatenate([xg_ref[:, s, :] for s in range(_SL)], axis=-1)
        mu = jnp.mean(xg, axis=-1, keepdims=True)
        var = jnp.mean((xg - mu) ** 2, axis=-1, keepdims=True)
        xn = (xg - mu) / jnp.sqrt(var + 1e-5) * lng_ref[0] + lnb_ref[0]
        h = jnp.dot(xn.astype(jnp.bfloat16), w1_ref[0],
                    preferred_element_type=jnp.float32) + b1_ref[0]
        h = jax.nn.gelu(h.astype(jnp.bfloat16))
        o = jnp.dot(h, w2_ref[0],
                    preferred_element_type=jnp.float32) + b2_ref[0]
        y = (xg + o) * gate_ref[...]
        for s in range(_SL):
            y_ref[:, s, :] = y[:, s * 128:(s + 1) * 128]


def _grouped_ffn(xg, row_gate, ln_g, ln_b, w1b, b1, w2b, b2,
                 block_expert, block_valid):
    grid_spec = pltpu.PrefetchScalarGridSpec(
        num_scalar_prefetch=2,
        grid=(G,),
        in_specs=[
            pl.BlockSpec((BLK, _SL, 128), lambda g, be, bv: (g, 0, 0)),
            pl.BlockSpec((BLK, 1), lambda g, be, bv: (g, 0)),
            pl.BlockSpec((1, 1, DIM), lambda g, be, bv: (be[g], 0, 0)),
            pl.BlockSpec((1, 1, DIM), lambda g, be, bv: (be[g], 0, 0)),
            pl.BlockSpec((1, DIM, MLP), lambda g, be, bv: (be[g], 0, 0)),
            pl.BlockSpec((1, 1, MLP), lambda g, be, bv: (be[g], 0, 0)),
            pl.BlockSpec((1, MLP, DIM), lambda g, be, bv: (be[g], 0, 0)),
            pl.BlockSpec((1, 1, DIM), lambda g, be, bv: (be[g], 0, 0)),
        ],
        out_specs=pl.BlockSpec((BLK, _SL, 128), lambda g, be, bv: (g, 0, 0)),
    )
    return pl.pallas_call(
        _ffn_block,
        grid_spec=grid_spec,
        out_shape=jax.ShapeDtypeStruct((P, _SL, 128), jnp.float32),
    )(block_expert, block_valid, xg, row_gate.reshape(P, 1),
      ln_g.reshape(E, 1, DIM), ln_b.reshape(E, 1, DIM), w1b,
      b1.reshape(E, 1, MLP), w2b, b2.reshape(E, 1, DIM))


_SL = 8
_LN = DIM // 128 // _SL


def _sc_gather(x3, row_token):
    rpw = P // NW
    ch = 16
    nch = rpw // ch
    nbuf = 4
    mesh = plsc.VectorSubcoreMesh(core_axis_name="c", subcore_axis_name="s")

    @functools.partial(
        pl.kernel, mesh=mesh,
        out_type=jax.ShapeDtypeStruct((P, _SL, 128), jnp.float32),
        compiler_params=pltpu.CompilerParams(use_tc_tiling_on_sc=True),
        scratch_types=[
            pltpu.VMEM((rpw,), jnp.int32),
        ] + [pltpu.VMEM((ch, _SL, 128), jnp.float32) for _ in range(nbuf)]
          + [pltpu.SemaphoreType.DMA for _ in range(2 * nbuf)],
    )
    def k(x_hbm, tok_hbm, out_hbm, idx_all, *bufs_sems):
        bufs = bufs_sems[:nbuf]
        gs = bufs_sems[nbuf:2 * nbuf]
        ss = bufs_sems[2 * nbuf:]
        wid = lax.axis_index("s") * _NC + lax.axis_index("c")
        base = wid * rpw
        pltpu.sync_copy(tok_hbm.at[pl.ds(base, rpw)], idx_all)

        def fire(c, b):
            return [pltpu.async_copy(
                        x_hbm.at[idx_all[pl.ds(c * ch + 16 * v, 16)]],
                        bufs[b].at[pl.ds(16 * v, 16)], gs[b])
                    for v in range(ch // 16)]

        gh = [fire(c, c) for c in range(nbuf)]
        sh = [None] * nbuf
        for c in range(nch):
            b = c % nbuf
            for h in gh[b]:
                h.wait()
            sh[b] = pltpu.async_copy(bufs[b], out_hbm.at[pl.ds(base + c * ch, ch)],
                                     ss[b])
            if c + nbuf < nch:
                sh[b].wait()
                gh[b] = fire(c + nbuf, b)
        for b in range(nbuf):
            sh[b].wait()

    return k(x3, row_token)


def _sc_combine(y3, pos0, pos1, n_tokens):
    rpw = n_tokens // NW
    ch = 16
    nch = rpw // ch
    mesh = plsc.VectorSubcoreMesh(core_axis_name="c", subcore_axis_name="s")

    @functools.partial(
        pl.kernel, mesh=mesh,
        out_type=jax.ShapeDtypeStruct((n_tokens, _SL, 128), jnp.float32),
        compiler_params=pltpu.CompilerParams(use_tc_tiling_on_sc=True),
        scratch_types=[
            pltpu.VMEM((rpw,), jnp.int32),
            pltpu.VMEM((rpw,), jnp.int32),
        ] + [pltpu.VMEM((ch, _SL, 128), jnp.float32) for _ in range(4)]
          + [pltpu.SemaphoreType.DMA for _ in range(6)],
    )
    def k(y_hbm, p0_hbm, p1_hbm, out_hbm, i0_v, i1_v, a0, a1, b0, b1,
          ga0, ga1, gb0, gb1, sa0, sa1):
        av, bv = [a0, a1], [b0, b1]
        gas, gbs, sas = [ga0, ga1], [gb0, gb1], [sa0, sa1]
        wid = lax.axis_index("s") * _NC + lax.axis_index("c")
        base = wid * rpw
        pltpu.sync_copy(p0_hbm.at[pl.ds(base, rpw)], i0_v)
        pltpu.sync_copy(p1_hbm.at[pl.ds(base, rpw)], i1_v)
        gha = [pltpu.async_copy(y_hbm.at[i0_v.at[pl.ds(c * ch, ch)]],
                                av[c], gas[c]) for c in range(2)]
        ghb = [pltpu.async_copy(y_hbm.at[i1_v.at[pl.ds(c * ch, ch)]],
                                bv[c], gbs[c]) for c in range(2)]
        sha = [None, None]
        for c in range(nch):
            s = c % 2
            gha[s].wait()
            ghb[s].wait()

            def row_add(r, _, _a=av[s], _b=bv[s]):
                for sub in range(_SL):
                    for j in range(8):
                        sl = pl.ds(j * 16, 16)
                        _a[r, sub, sl] = _a[r, sub, sl] + _b[r, sub, sl]
                return 0

            lax.fori_loop(0, ch, row_add, 0)
            sha[s] = pltpu.async_copy(av[s], out_hbm.at[pl.ds(base + c * ch, ch)],
                                      sas[s])
            if c + 2 < nch:
                sha[s].wait()
                gha[s] = pltpu.async_copy(
                    y_hbm.at[i0_v.at[pl.ds((c + 2) * ch, ch)]], av[s], gas[s])
                ghb[s] = pltpu.async_copy(
                    y_hbm.at[i1_v.at[pl.ds((c + 2) * ch, ch)]], bv[s], gbs[s])
        for s in range(2):
            sha[s].wait()

    return k(y3, pos0, pos1)


def _route(xf, Wr):
    n = xf.shape[0]
    nk = n * TOPK
    logits = xf @ Wr
    probs = jax.nn.softmax(logits, axis=-1)
    topv, topi = jax.lax.top_k(probs, TOPK)
    gates = topv / (jnp.sum(topv, axis=-1, keepdims=True) + 1e-9)

    e_flat = topi.reshape(-1).astype(jnp.int32)
    order = jnp.argsort(e_flat, stable=True).astype(jnp.int32)
    e_sorted = e_flat[order]
    counts = jnp.bincount(e_flat, length=E).astype(jnp.int32)
    blocks_e = (counts + BLK - 1) // BLK
    bcum = jnp.cumsum(blocks_e)
    padded_start = (bcum - blocks_e) * BLK
    csum = jnp.cumsum(counts) - counts
    ranks = jnp.arange(nk, dtype=jnp.int32) - csum[e_sorted]
    p_sorted = (padded_start[e_sorted] + ranks).astype(jnp.int32)
    tok_sorted = (order // TOPK).astype(jnp.int32)

    row_token = jnp.zeros((P,), jnp.int32).at[p_sorted].set(tok_sorted)
    row_gate = jnp.zeros((P,), jnp.float32).at[p_sorted].set(
        gates.reshape(-1)[order])
    pos = jnp.zeros((nk,), jnp.int32).at[order].set(p_sorted).reshape(n, TOPK)

    garr = jnp.arange(G, dtype=jnp.int32)
    block_expert_raw = jnp.searchsorted(bcum, garr, side="right").astype(jnp.int32)
    last_e = jnp.argmax(jnp.where(counts > 0,
                                  jnp.arange(E, dtype=jnp.int32), -1)).astype(jnp.int32)
    block_expert = jnp.minimum(block_expert_raw, last_e)
    block_valid = (garr < bcum[-1]).astype(jnp.int32)
    return row_token, row_gate, pos, block_expert, block_valid


def kernel(x, Wr, ln_g, ln_b, W1, b1, W2, b2):
    b, t, d = x.shape
    n = b * t
    xf = x.reshape(n, d)

    row_token, row_gate, pos, block_expert, block_valid = _route(xf, Wr)

    x3 = _linearize(xf, n)
    xg3 = _sc_gather(x3, row_token)

    w1b = W1.astype(jnp.bfloat16)
    w2b = W2.astype(jnp.bfloat16)
    y3 = _grouped_ffn(xg3, row_gate, ln_g, ln_b, w1b, b1, w2b, b2,
                      block_expert, block_valid)

    out = _sc_combine(y3, pos[:, 0], pos[:, 1], n)
    return out.reshape(b, t, d)


# speedup vs baseline: 1.2567x; 1.2567x over previous
import functools

import jax
import jax.numpy as jnp
from jax import lax
from jax.experimental import pallas as pl
from jax.experimental.pallas import tpu as pltpu

try:
    from jax.experimental.pallas import tpu_sc as plsc
    _HAS_SC = True
except ImportError:
    plsc = None
    _HAS_SC = False

DIM = 1024
MLP = 4096
E = 8
TOPK = 2
BLK = 256
G = 16 + E
P = G * BLK

NW = 32
_NC = 2


def _lin_block(x_ref, x3_ref):
    for s in range(_SL):
        x3_ref[:, s, :] = x_ref[:, s * 128:(s + 1) * 128]


def _linearize(xf, n):
    blk = 256
    return pl.pallas_call(
        _lin_block,
        grid=(n // blk,),
        in_specs=[pl.BlockSpec((blk, DIM), lambda i: (i, 0))],
        out_specs=pl.BlockSpec((blk, _SL, 128), lambda i: (i, 0, 0)),
        out_shape=jax.ShapeDtypeStruct((n, _SL, 128), jnp.float32),
    )(xf)


def _ffn_block(be_ref, bv_ref, xg_ref, gate_ref, lng_ref, lnb_ref,
               w1_ref, b1_ref, w2_ref, b2_ref, y_ref):
    g = pl.program_id(0)

    @pl.when(bv_ref[g] != 0)
    def _():
        xg = jnp.concatenate([xg_ref[:, s, :] for s in range(_SL)], axis=-1)
        mu = jnp.mean(xg, axis=-1, keepdims=True)
        var = jnp.mean((xg - mu) ** 2, axis=-1, keepdims=True)
        xn = (xg - mu) / jnp.sqrt(var + 1e-5) * lng_ref[0] + lnb_ref[0]
        h = jnp.dot(xn.astype(jnp.bfloat16), w1_ref[0],
                    preferred_element_type=jnp.float32) + b1_ref[0]
        h = jax.nn.gelu(h.astype(jnp.bfloat16))
        o = jnp.dot(h, w2_ref[0],
                    preferred_element_type=jnp.float32) + b2_ref[0]
        y = (xg + o) * gate_ref[...]
        for s in range(_SL):
            y_ref[:, s, :] = y[:, s * 128:(s + 1) * 128]


def _grouped_ffn(xg, row_gate, ln_g, ln_b, w1b, b1, w2b, b2,
                 block_expert, block_valid):
    grid_spec = pltpu.PrefetchScalarGridSpec(
        num_scalar_prefetch=2,
        grid=(G,),
        in_specs=[
            pl.BlockSpec((BLK, _SL, 128), lambda g, be, bv: (g, 0, 0)),
            pl.BlockSpec((BLK, 1), lambda g, be, bv: (g, 0)),
            pl.BlockSpec((1, 1, DIM), lambda g, be, bv: (be[g], 0, 0)),
            pl.BlockSpec((1, 1, DIM), lambda g, be, bv: (be[g], 0, 0)),
            pl.BlockSpec((1, DIM, MLP), lambda g, be, bv: (be[g], 0, 0)),
            pl.BlockSpec((1, 1, MLP), lambda g, be, bv: (be[g], 0, 0)),
            pl.BlockSpec((1, MLP, DIM), lambda g, be, bv: (be[g], 0, 0)),
            pl.BlockSpec((1, 1, DIM), lambda g, be, bv: (be[g], 0, 0)),
        ],
        out_specs=pl.BlockSpec((BLK, _SL, 128), lambda g, be, bv: (g, 0, 0)),
    )
    return pl.pallas_call(
        _ffn_block,
        grid_spec=grid_spec,
        out_shape=jax.ShapeDtypeStruct((P, _SL, 128), jnp.float32),
    )(block_expert, block_valid, xg, row_gate.reshape(P, 1),
      ln_g.reshape(E, 1, DIM), ln_b.reshape(E, 1, DIM), w1b,
      b1.reshape(E, 1, MLP), w2b, b2.reshape(E, 1, DIM))


_SL = 8
_LN = DIM // 128 // _SL


def _sc_gather(x3, row_token):
    rpw = P // NW
    ch = 16
    nch = rpw // ch
    nbuf = 4
    mesh = plsc.VectorSubcoreMesh(core_axis_name="c", subcore_axis_name="s")

    @functools.partial(
        pl.kernel, mesh=mesh,
        out_type=jax.ShapeDtypeStruct((P, _SL, 128), jnp.float32),
        compiler_params=pltpu.CompilerParams(use_tc_tiling_on_sc=True),
        scratch_types=[
            pltpu.VMEM((rpw,), jnp.int32),
        ] + [pltpu.VMEM((ch, _SL, 128), jnp.float32) for _ in range(nbuf)]
          + [pltpu.SemaphoreType.DMA for _ in range(2 * nbuf)],
    )
    def k(x_hbm, tok_hbm, out_hbm, idx_all, *bufs_sems):
        bufs = bufs_sems[:nbuf]
        gs = bufs_sems[nbuf:2 * nbuf]
        ss = bufs_sems[2 * nbuf:]
        wid = lax.axis_index("s") * _NC + lax.axis_index("c")
        base = wid * rpw
        pltpu.sync_copy(tok_hbm.at[pl.ds(base, rpw)], idx_all)

        def fire(c, b):
            return [pltpu.async_copy(
                        x_hbm.at[idx_all[pl.ds(c * ch + 16 * v, 16)]],
                        bufs[b].at[pl.ds(16 * v, 16)], gs[b])
                    for v in range(ch // 16)]

        gh = [fire(c, c) for c in range(nbuf)]
        sh = [None] * nbuf
        for c in range(nch):
            b = c % nbuf
            for h in gh[b]:
                h.wait()
            sh[b] = pltpu.async_copy(bufs[b], out_hbm.at[pl.ds(base + c * ch, ch)],
                                     ss[b])
            if c + nbuf < nch:
                sh[b].wait()
                gh[b] = fire(c + nbuf, b)
        for b in range(nbuf):
            sh[b].wait()

    return k(x3, row_token)


def _sc_combine(y3, pos0, pos1, n_tokens):
    rpw = n_tokens // NW
    ch = 16
    nch = rpw // ch
    mesh = plsc.VectorSubcoreMesh(core_axis_name="c", subcore_axis_name="s")

    @functools.partial(
        pl.kernel, mesh=mesh,
        out_type=jax.ShapeDtypeStruct((n_tokens, _SL, 128), jnp.float32),
        compiler_params=pltpu.CompilerParams(use_tc_tiling_on_sc=True),
        scratch_types=[
            pltpu.VMEM((rpw,), jnp.int32),
            pltpu.VMEM((rpw,), jnp.int32),
        ] + [pltpu.VMEM((ch, _SL, 128), jnp.float32) for _ in range(4)]
          + [pltpu.SemaphoreType.DMA for _ in range(6)],
    )
    def k(y_hbm, p0_hbm, p1_hbm, out_hbm, i0_v, i1_v, a0, a1, b0, b1,
          ga0, ga1, gb0, gb1, sa0, sa1):
        av, bv = [a0, a1], [b0, b1]
        gas, gbs, sas = [ga0, ga1], [gb0, gb1], [sa0, sa1]
        wid = lax.axis_index("s") * _NC + lax.axis_index("c")
        base = wid * rpw
        pltpu.sync_copy(p0_hbm.at[pl.ds(base, rpw)], i0_v)
        pltpu.sync_copy(p1_hbm.at[pl.ds(base, rpw)], i1_v)
        gha = [pltpu.async_copy(y_hbm.at[i0_v.at[pl.ds(c * ch, ch)]],
                                av[c], gas[c]) for c in range(2)]
        ghb = [pltpu.async_copy(y_hbm.at[i1_v.at[pl.ds(c * ch, ch)]],
                                bv[c], gbs[c]) for c in range(2)]
        sha = [None, None]
        for c in range(nch):
            s = c % 2
            gha[s].wait()
            ghb[s].wait()

            def row_add(r, _, _a=av[s], _b=bv[s]):
                for sub in range(_SL):
                    for j in range(8):
                        sl = pl.ds(j * 16, 16)
                        _a[r, sub, sl] = _a[r, sub, sl] + _b[r, sub, sl]
                return 0

            lax.fori_loop(0, ch, row_add, 0)
            sha[s] = pltpu.async_copy(av[s], out_hbm.at[pl.ds(base + c * ch, ch)],
                                      sas[s])
            if c + 2 < nch:
                sha[s].wait()
                gha[s] = pltpu.async_copy(
                    y_hbm.at[i0_v.at[pl.ds((c + 2) * ch, ch)]], av[s], gas[s])
                ghb[s] = pltpu.async_copy(
                    y_hbm.at[i1_v.at[pl.ds((c + 2) * ch, ch)]], bv[s], gbs[s])
        for s in range(2):
            sha[s].wait()

    return k(y3, pos0, pos1)


def _route(xf, Wr):
    n = xf.shape[0]
    nk = n * TOPK
    logits = xf @ Wr
    probs = jax.nn.softmax(logits, axis=-1)
    topv, topi = jax.lax.top_k(probs, TOPK)
    gates = topv / (jnp.sum(topv, axis=-1, keepdims=True) + 1e-9)

    e_flat = topi.reshape(-1).astype(jnp.int32)
    order = jnp.argsort(e_flat, stable=True).astype(jnp.int32)
    e_sorted = e_flat[order]
    counts = jnp.bincount(e_flat, length=E).astype(jnp.int32)
    blocks_e = (counts + BLK - 1) // BLK
    bcum = jnp.cumsum(blocks_e)
    padded_start = (bcum - blocks_e) * BLK
    csum = jnp.cumsum(counts) - counts
    ranks = jnp.arange(nk, dtype=jnp.int32) - csum[e_sorted]
    p_sorted = (padded_start[e_sorted] + ranks).astype(jnp.int32)
    tok_sorted = (order // TOPK).astype(jnp.int32)

    row_token = (jnp.arange(P, dtype=jnp.int32) % n).at[p_sorted].set(tok_sorted)
    row_gate = jnp.zeros((P,), jnp.float32).at[p_sorted].set(
        gates.reshape(-1)[order])
    pos = jnp.zeros((nk,), jnp.int32).at[order].set(p_sorted).reshape(n, TOPK)

    garr = jnp.arange(G, dtype=jnp.int32)
    block_expert_raw = jnp.searchsorted(bcum, garr, side="right").astype(jnp.int32)
    last_e = jnp.argmax(jnp.where(counts > 0,
                                  jnp.arange(E, dtype=jnp.int32), -1)).astype(jnp.int32)
    block_expert = jnp.minimum(block_expert_raw, last_e)
    block_valid = (garr < bcum[-1]).astype(jnp.int32)
    return row_token, row_gate, pos, block_expert, block_valid


def kernel(x, Wr, ln_g, ln_b, W1, b1, W2, b2):
    b, t, d = x.shape
    n = b * t
    xf = x.reshape(n, d)

    row_token, row_gate, pos, block_expert, block_valid = _route(xf, Wr)

    x3 = _linearize(xf, n)
    xg3 = _sc_gather(x3, row_token)

    w1b = W1.astype(jnp.bfloat16)
    w2b = W2.astype(jnp.bfloat16)
    y3 = _grouped_ffn(xg3, row_gate, ln_g, ln_b, w1b, b1, w2b, b2,
                      block_expert, block_valid)

    out = _sc_combine(y3, pos[:, 0], pos[:, 1], n)
    return out.reshape(b, t, d)


# speedup vs baseline: 1.3583x; 1.0808x over previous
import functools

import jax
import jax.numpy as jnp
from jax import lax
from jax.experimental import pallas as pl
from jax.experimental.pallas import tpu as pltpu

try:
    from jax.experimental.pallas import tpu_sc as plsc
    _HAS_SC = True
except ImportError:
    plsc = None
    _HAS_SC = False

DIM = 1024
MLP = 4096
E = 8
TOPK = 2
BLK = 256
G = 16 + E
P = G * BLK

NW = 32
_NC = 2


def _lin_block(x_ref, x3_ref):
    for s in range(_SL):
        x3_ref[:, s, :] = x_ref[:, s * 128:(s + 1) * 128]


def _linearize(xf, n):
    blk = 256
    return pl.pallas_call(
        _lin_block,
        grid=(n // blk,),
        in_specs=[pl.BlockSpec((blk, DIM), lambda i: (i, 0))],
        out_specs=pl.BlockSpec((blk, _SL, 128), lambda i: (i, 0, 0)),
        out_shape=jax.ShapeDtypeStruct((n, _SL, 128), jnp.float32),
    )(xf)


def _ffn_block(be_ref, bv_ref, xg_ref, gate_ref, lng_ref, lnb_ref,
               w1_ref, b1_ref, w2_ref, b2_ref, y_ref):
    g = pl.program_id(0)

    @pl.when(bv_ref[g] != 0)
    def _():
        xg = jnp.concatenate([xg_ref[:, s, :] for s in range(_SL)], axis=-1)
        mu = jnp.mean(xg, axis=-1, keepdims=True)
        var = jnp.mean((xg - mu) ** 2, axis=-1, keepdims=True)
        xn = (xg - mu) / jnp.sqrt(var + 1e-5) * lng_ref[0] + lnb_ref[0]
        h = jnp.dot(xn.astype(jnp.bfloat16), w1_ref[0],
                    preferred_element_type=jnp.float32) + b1_ref[0]
        h = jax.nn.gelu(h.astype(jnp.bfloat16))
        o = jnp.dot(h, w2_ref[0],
                    preferred_element_type=jnp.float32) + b2_ref[0]
        y = (xg + o) * gate_ref[...]
        for s in range(_SL):
            y_ref[:, s, :] = y[:, s * 128:(s + 1) * 128]


def _grouped_ffn(xg, row_gate, ln_g, ln_b, w1b, b1, w2b, b2,
                 block_expert, block_valid):
    grid_spec = pltpu.PrefetchScalarGridSpec(
        num_scalar_prefetch=2,
        grid=(G,),
        in_specs=[
            pl.BlockSpec((BLK, _SL, 128), lambda g, be, bv: (g, 0, 0)),
            pl.BlockSpec((BLK, 1), lambda g, be, bv: (g, 0)),
            pl.BlockSpec((1, 1, DIM), lambda g, be, bv: (be[g], 0, 0)),
            pl.BlockSpec((1, 1, DIM), lambda g, be, bv: (be[g], 0, 0)),
            pl.BlockSpec((1, DIM, MLP), lambda g, be, bv: (be[g], 0, 0)),
            pl.BlockSpec((1, 1, MLP), lambda g, be, bv: (be[g], 0, 0)),
            pl.BlockSpec((1, MLP, DIM), lambda g, be, bv: (be[g], 0, 0)),
            pl.BlockSpec((1, 1, DIM), lambda g, be, bv: (be[g], 0, 0)),
        ],
        out_specs=pl.BlockSpec((BLK, _SL, 128), lambda g, be, bv: (g, 0, 0)),
    )
    return pl.pallas_call(
        _ffn_block,
        grid_spec=grid_spec,
        out_shape=jax.ShapeDtypeStruct((P, _SL, 128), jnp.float32),
    )(block_expert, block_valid, xg, row_gate.reshape(P, 1),
      ln_g.reshape(E, 1, DIM), ln_b.reshape(E, 1, DIM), w1b,
      b1.reshape(E, 1, MLP), w2b, b2.reshape(E, 1, DIM))


_SL = 8
_LN = DIM // 128 // _SL


def _sc_gather(x3, row_token):
    rpw = P // NW
    ch = 16
    nch = rpw // ch
    nbuf = 4
    mesh = plsc.VectorSubcoreMesh(core_axis_name="c", subcore_axis_name="s")

    @functools.partial(
        pl.kernel, mesh=mesh,
        out_type=jax.ShapeDtypeStruct((P, _SL, 128), jnp.float32),
        compiler_params=pltpu.CompilerParams(use_tc_tiling_on_sc=True),
        scratch_types=[
            pltpu.VMEM((rpw,), jnp.int32),
        ] + [pltpu.VMEM((ch, _SL, 128), jnp.float32) for _ in range(nbuf)]
          + [pltpu.SemaphoreType.DMA for _ in range(2 * nbuf)],
    )
    def k(x_hbm, tok_hbm, out_hbm, idx_all, *bufs_sems):
        bufs = bufs_sems[:nbuf]
        gs = bufs_sems[nbuf:2 * nbuf]
        ss = bufs_sems[2 * nbuf:]
        wid = lax.axis_index("s") * _NC + lax.axis_index("c")
        base = wid * rpw
        pltpu.sync_copy(tok_hbm.at[pl.ds(base, rpw)], idx_all)

        def fire(c, b):
            return [pltpu.async_copy(
                        x_hbm.at[idx_all[pl.ds(c * ch + 16 * v, 16)]],
                        bufs[b].at[pl.ds(16 * v, 16)], gs[b])
                    for v in range(ch // 16)]

        gh = [fire(c, c) for c in range(nbuf)]
        sh = [None] * nbuf
        for c in range(nch):
            b = c % nbuf
            for h in gh[b]:
                h.wait()
            sh[b] = pltpu.async_copy(bufs[b], out_hbm.at[pl.ds(base + c * ch, ch)],
                                     ss[b])
            if c + nbuf < nch:
                sh[b].wait()
                gh[b] = fire(c + nbuf, b)
        for b in range(nbuf):
            sh[b].wait()

    return k(x3, row_token)


def _sc_combine(y3, pos0, pos1, n_tokens):
    rpw = n_tokens // NW
    ch = 16
    nch = rpw // ch
    mesh = plsc.VectorSubcoreMesh(core_axis_name="c", subcore_axis_name="s")

    @functools.partial(
        pl.kernel, mesh=mesh,
        out_type=jax.ShapeDtypeStruct((n_tokens, _SL, 128), jnp.float32),
        compiler_params=pltpu.CompilerParams(use_tc_tiling_on_sc=True),
        scratch_types=[
            pltpu.VMEM((rpw,), jnp.int32),
            pltpu.VMEM((rpw,), jnp.int32),
        ] + [pltpu.VMEM((ch, _SL, 128), jnp.float32) for _ in range(4)]
          + [pltpu.SemaphoreType.DMA for _ in range(6)],
    )
    def k(y_hbm, p0_hbm, p1_hbm, out_hbm, i0_v, i1_v, a0, a1, b0, b1,
          ga0, ga1, gb0, gb1, sa0, sa1):
        av, bv = [a0, a1], [b0, b1]
        gas, gbs, sas = [ga0, ga1], [gb0, gb1], [sa0, sa1]
        wid = lax.axis_index("s") * _NC + lax.axis_index("c")
        base = wid * rpw
        pltpu.sync_copy(p0_hbm.at[pl.ds(base, rpw)], i0_v)
        pltpu.sync_copy(p1_hbm.at[pl.ds(base, rpw)], i1_v)
        gha = [pltpu.async_copy(y_hbm.at[i0_v.at[pl.ds(c * ch, ch)]],
                                av[c], gas[c]) for c in range(2)]
        ghb = [pltpu.async_copy(y_hbm.at[i1_v.at[pl.ds(c * ch, ch)]],
                                bv[c], gbs[c]) for c in range(2)]
        sha = [None, None]
        for c in range(nch):
            s = c % 2
            gha[s].wait()
            ghb[s].wait()

            def row_add(r, _, _a=av[s], _b=bv[s]):
                for sub in range(_SL):
                    for j in range(8):
                        sl = pl.ds(j * 16, 16)
                        _a[r, sub, sl] = _a[r, sub, sl] + _b[r, sub, sl]
                return 0

            lax.fori_loop(0, ch, row_add, 0)
            sha[s] = pltpu.async_copy(av[s], out_hbm.at[pl.ds(base + c * ch, ch)],
                                      sas[s])
            if c + 2 < nch:
                sha[s].wait()
                gha[s] = pltpu.async_copy(
                    y_hbm.at[i0_v.at[pl.ds((c + 2) * ch, ch)]], av[s], gas[s])
                ghb[s] = pltpu.async_copy(
                    y_hbm.at[i1_v.at[pl.ds((c + 2) * ch, ch)]], bv[s], gbs[s])
        for s in range(2):
            sha[s].wait()

    return k(y3, pos0, pos1)


def _route(xf, Wr):
    n = xf.shape[0]
    nk = n * TOPK
    logits = xf @ Wr
    probs = jax.nn.softmax(logits, axis=-1)
    topv, topi = jax.lax.top_k(probs, TOPK)
    gates = topv / (jnp.sum(topv, axis=-1, keepdims=True) + 1e-9)

    e_flat = topi.reshape(-1).astype(jnp.int32)
    oh = jax.nn.one_hot(e_flat, E, dtype=jnp.int32)
    ranks_all = jnp.cumsum(oh, axis=0) - oh
    rank = jnp.take_along_axis(ranks_all, e_flat[:, None], axis=1)[:, 0]
    counts = jnp.sum(oh, axis=0).astype(jnp.int32)
    blocks_e = (counts + BLK - 1) // BLK
    bcum = jnp.cumsum(blocks_e)
    padded_start = (bcum - blocks_e) * BLK
    p_flat = (padded_start[e_flat] + rank).astype(jnp.int32)

    row_token = (jnp.arange(P, dtype=jnp.int32) % n).at[p_flat].set(
        jnp.arange(nk, dtype=jnp.int32) // TOPK)
    row_gate = jnp.zeros((P,), jnp.float32).at[p_flat].set(gates.reshape(-1))
    pos = p_flat.reshape(n, TOPK)

    garr = jnp.arange(G, dtype=jnp.int32)
    block_expert_raw = jnp.searchsorted(bcum, garr, side="right").astype(jnp.int32)
    last_e = jnp.argmax(jnp.where(counts > 0,
                                  jnp.arange(E, dtype=jnp.int32), -1)).astype(jnp.int32)
    block_expert = jnp.minimum(block_expert_raw, last_e)
    block_valid = (garr < bcum[-1]).astype(jnp.int32)
    return row_token, row_gate, pos, block_expert, block_valid


def kernel(x, Wr, ln_g, ln_b, W1, b1, W2, b2):
    b, t, d = x.shape
    n = b * t
    xf = x.reshape(n, d)

    row_token, row_gate, pos, block_expert, block_valid = _route(xf, Wr)

    x3 = _linearize(xf, n)
    xg3 = _sc_gather(x3, row_token)

    w1b = W1.astype(jnp.bfloat16)
    w2b = W2.astype(jnp.bfloat16)
    y3 = _grouped_ffn(xg3, row_gate, ln_g, ln_b, w1b, b1, w2b, b2,
                      block_expert, block_valid)

    out = _sc_combine(y3, pos[:, 0], pos[:, 1], n)
    return out.reshape(b, t, d)
